# baseline (speedup 1.0000x reference)
import math

import jax
import jax.numpy as jnp
from jax.experimental import pallas as pl
from jax.experimental.pallas import tpu as pltpu

_BN_EPS = 1e-5
_H = 28
_W = 28
_WP = 32
_IMG = 31 * _WP
_OUT = _H * _WP
_B = 8


def _cparams():
    return pltpu.CompilerParams(
        dimension_semantics=("arbitrary",),
        vmem_limit_bytes=100 * 1024 * 1024,
    )


def _shifted_copies(xb):
    rows = xb.shape[0]
    zt = jnp.zeros((66, 128), dtype=xb.dtype)
    ext = jnp.concatenate([xb, zt], axis=0)
    return [ext[0:rows + 64], ext[1:rows + 65], ext[2:rows + 66]]


def _conv9(cs, w_ref, B):
    wk = w_ref[...].reshape(9 * 128, 128).astype(jnp.bfloat16)
    accs = []
    for i in range(B):
        base = i * _IMG
        patches = [cs[k % 3][base + (k // 3) * _WP:
                             base + (k // 3) * _WP + _OUT]
                   for k in range(9)]
        p = jnp.concatenate(patches, axis=1)
        accs.append(jnp.dot(p, wk, preferred_element_type=jnp.float32))
    return jnp.stack(accs, axis=0)


def _affine_from_stats(s_ref, q_ref, g_ref, b_ref):
    m = float(s_ref.shape[0] * _H * _W)
    total = jnp.sum(s_ref[:, 0, :], axis=0, keepdims=True)
    totsq = jnp.sum(q_ref[:, 0, :], axis=0, keepdims=True)
    mean = total / m
    var = totsq / m - mean * mean
    scale = g_ref[...] * jax.lax.rsqrt(var + _BN_EPS)
    shift = b_ref[...] - mean * scale
    return scale, shift


def _finish(acc, y_ref, ssum_ref, ssq_ref):
    B = y_ref.shape[0]
    y_ref[...] = acc.astype(jnp.bfloat16)
    a4 = acc.reshape(B, _H, _WP, 128)
    s32 = jnp.sum(a4, axis=1)
    q32 = jnp.sum(a4 * a4, axis=1)
    ssum_ref[:, 0, :] = jnp.sum(s32[:, :_W, :], axis=1)
    ssq_ref[:, 0, :] = jnp.sum(q32[:, :_W, :], axis=1)


def _conv1_kernel(xr_ref, w_ref, y_ref, ssum_ref, ssq_ref):
    B = xr_ref.shape[0]
    xt = jnp.transpose(xr_ref[...], (0, 2, 1))
    x4 = jnp.pad(xt.reshape(B, _H, _W, 128),
                 ((0, 0), (0, 0), (1, 3), (0, 0)))
    xf = x4.reshape(B, _OUT, 128)
    zt = jnp.zeros((B, _WP, 128), dtype=jnp.bfloat16)
    zb = jnp.zeros((B, 2 * _WP, 128), dtype=jnp.bfloat16)
    xb = jnp.concatenate([zt, xf, zb], axis=1).reshape(B * _IMG, 128)
    acc = _conv9(_shifted_copies(xb), w_ref, B)
    _finish(acc, y_ref, ssum_ref, ssq_ref)


def _conv2_kernel(y1_ref, s_ref, q_ref, g_ref, b_ref, w_ref,
                  y2_ref, ssum_ref, ssq_ref):
    sc, sh = _affine_from_stats(s_ref, q_ref, g_ref, b_ref)
    a = y1_ref[...].astype(jnp.float32) * sc + sh
    a = jnp.maximum(a, 0.0)

    B = y1_ref.shape[0]
    m32 = jax.lax.broadcasted_iota(jnp.int32, (_WP, 128), 0) < _W
    a4 = a.reshape(B, _H, _WP, 128)
    ab = jnp.where(m32[None, None, :, :], a4, 0.0).astype(jnp.bfloat16)
    ab = ab.reshape(B, _OUT, 128)
    zt64 = jnp.zeros((64, 128), dtype=jnp.bfloat16)

    def _placed(base):
        zt = jnp.zeros((B, base, 128), dtype=jnp.bfloat16)
        zb = jnp.zeros((B, _IMG - _OUT - base, 128), dtype=jnp.bfloat16)
        flat = jnp.concatenate([zt, ab, zb], axis=1).reshape(B * _IMG, 128)
        return jnp.concatenate([flat, zt64], axis=0)

    cs = [_placed(33), _placed(32), _placed(31)]
    acc = _conv9(cs, w_ref, B)
    _finish(acc, y2_ref, ssum_ref, ssq_ref)


def _out_kernel(y2_ref, s_ref, q_ref, g_ref, b_ref, xr_ref, o_ref):
    sc, sh = _affine_from_stats(s_ref, q_ref, g_ref, b_ref)
    B = y2_ref.shape[0]
    v = y2_ref[...].astype(jnp.float32) * sc + sh
    v = v.reshape(B, _H, _WP, 128)[:, :, :_W, :].reshape(B, _H * _W, 128)
    vt = jnp.transpose(v, (0, 2, 1))
    o_ref[...] = jnp.maximum(vt + xr_ref[...].astype(jnp.float32), 0.0)


def _conv1_call(xr, w1m):
    N = xr.shape[0]
    bb = math.gcd(N, _B)
    G = N // bb
    flops = 2 * N * _H * _W * 9 * 128 * 128
    return pl.pallas_call(
        _conv1_kernel,
        out_shape=(jax.ShapeDtypeStruct((N, _OUT, 128), jnp.bfloat16),
                   jax.ShapeDtypeStruct((N, 1, 128), jnp.float32),
                   jax.ShapeDtypeStruct((N, 1, 128), jnp.float32)),
        grid=(G,),
        in_specs=[pl.BlockSpec((bb, 128, _H * _W), lambda n: (n, 0, 0)),
                  pl.BlockSpec((9, 128, 128), lambda n: (0, 0, 0))],
        out_specs=(pl.BlockSpec((bb, _OUT, 128), lambda n: (n, 0, 0)),
                   pl.BlockSpec((bb, 1, 128), lambda n: (n, 0, 0)),
                   pl.BlockSpec((bb, 1, 128), lambda n: (n, 0, 0))),
        compiler_params=_cparams(),
        cost_estimate=pl.CostEstimate(
            flops=flops, transcendentals=0,
            bytes_accessed=4 * xr.size + 2 * N * _OUT * 128),
    )(xr, w1m)


def _conv2_call(y1, s1, q1, g1, b1, w2m):
    N = y1.shape[0]
    bb = math.gcd(N, _B)
    G = N // bb
    flops = 2 * N * _H * _W * 9 * 128 * 128
    return pl.pallas_call(
        _conv2_kernel,
        out_shape=(jax.ShapeDtypeStruct((N, _OUT, 128), jnp.bfloat16),
                   jax.ShapeDtypeStruct((N, 1, 128), jnp.float32),
                   jax.ShapeDtypeStruct((N, 1, 128), jnp.float32)),
        grid=(G,),
        in_specs=[pl.BlockSpec((bb, _OUT, 128), lambda n: (n, 0, 0)),
                  pl.BlockSpec((N, 1, 128), lambda n: (0, 0, 0)),
                  pl.BlockSpec((N, 1, 128), lambda n: (0, 0, 0)),
                  pl.BlockSpec((1, 128), lambda n: (0, 0)),
                  pl.BlockSpec((1, 128), lambda n: (0, 0)),
                  pl.BlockSpec((9, 128, 128), lambda n: (0, 0, 0))],
        out_specs=(pl.BlockSpec((bb, _OUT, 128), lambda n: (n, 0, 0)),
                   pl.BlockSpec((bb, 1, 128), lambda n: (n, 0, 0)),
                   pl.BlockSpec((bb, 1, 128), lambda n: (n, 0, 0))),
        compiler_params=_cparams(),
        cost_estimate=pl.CostEstimate(
            flops=flops, transcendentals=0,
            bytes_accessed=4 * N * _OUT * 128),
    )(y1, s1, q1, g1, b1, w2m)


def _out_call(y2, s2, q2, g2, b2, xr):
    N = y2.shape[0]
    bb = math.gcd(N, _B)
    G = N // bb
    return pl.pallas_call(
        _out_kernel,
        out_shape=jax.ShapeDtypeStruct((N, 128, _H * _W), jnp.float32),
        grid=(G,),
        in_specs=[pl.BlockSpec((bb, _OUT, 128), lambda n: (n, 0, 0)),
                  pl.BlockSpec((N, 1, 128), lambda n: (0, 0, 0)),
                  pl.BlockSpec((N, 1, 128), lambda n: (0, 0, 0)),
                  pl.BlockSpec((1, 128), lambda n: (0, 0)),
                  pl.BlockSpec((1, 128), lambda n: (0, 0)),
                  pl.BlockSpec((bb, 128, _H * _W), lambda n: (n, 0, 0))],
        out_specs=pl.BlockSpec((bb, 128, _H * _W), lambda n: (n, 0, 0)),
        compiler_params=_cparams(),
        cost_estimate=pl.CostEstimate(
            flops=4 * N * _OUT * 128, transcendentals=0,
            bytes_accessed=12 * N * _H * _W * 128),
    )(y2, s2, q2, g2, b2, xr)


@jax.jit
def _forward(x_nchw, w1m, g1, b1, w2m, g2, b2):
    N, C, H, W = x_nchw.shape
    xr = x_nchw.astype(jnp.bfloat16).reshape(N, C, H * W)
    g1r, b1r = g1.reshape(1, C), b1.reshape(1, C)
    g2r, b2r = g2.reshape(1, C), b2.reshape(1, C)

    y1, s1, q1 = _conv1_call(xr, w1m)
    y2, s2, q2 = _conv2_call(y1, s1, q1, g1r, b1r, w2m)
    o = _out_call(y2, s2, q2, g2r, b2r, xr)
    return o.reshape(N, C, H, W)


def kernel(x_nchw, w1m, g1, b1, w2m, g2, b2):
    return _forward(x_nchw, w1m, g1, b1, w2m, g2, b2)
